# baseline (speedup 1.0000x reference)
import functools

import jax
import jax.numpy as jnp
from jax import lax
from jax.experimental import pallas as pl
from jax.experimental.pallas import tpu as pltpu
from jax.experimental.pallas import tpu_sc as plsc

B = 16384
D = 32
V = 1000000
NC = 2
NS = 16
NW = NC * NS
BPW = B // NW
CHUNK = 128
NCHUNK = BPW // CHUNK
WIN = 128
TAIL = V - (V // WIN) * WIN
TSTART = V - TAIL
RING = 8


def _win_start(r):
    return jnp.minimum((r // WIN) * WIN, TSTART - WIN)


def _gather_one(tab_t, tail_v, pk_v, orig_v, uniq_s, rows_v, bufs, sems):
    d16a = lax.iota(jnp.int32, 16)
    d16b = d16a + 16

    def rdt(i):
        pk = pk_v[pl.ds(i, 16)][0]
        b = pk >> 14
        return jnp.minimum(b * WIN, TSTART - WIN), pk & 16383

    def prepass(i, carry):
        prev_t, nu = carry
        t, _ = rdt(i)
        new = t != prev_t

        @pl.when(new)
        def _():
            uniq_s[nu] = t

        return t, nu + new.astype(jnp.int32)

    _, n_u = lax.fori_loop(0, BPW, prepass, (jnp.int32(-1), jnp.int32(0)))

    def issue(f, p):
        t = uniq_s[f]
        pltpu.async_copy(
            tab_t.at[:, pl.ds(pl.multiple_of(t, WIN), WIN)],
            bufs.at[p], sems.at[p])

    for f in range(RING - 1):
        @pl.when(f < n_u)
        def _(f=f):
            issue(f, f)

    def step(i, carry):
        prev_t, f = carry
        t, pos = rdt(i)
        r = orig_v[pl.ds(pos, 16)][0]
        new = t != prev_t
        f = f + new.astype(jnp.int32)
        p = lax.rem(f, RING)

        @pl.when(new)
        def _():
            pltpu.make_async_copy(
                tab_t.at[:, pl.ds(0, WIN)], bufs.at[p], sems.at[p]).wait()
            g = f + RING - 1

            @pl.when(g < n_u)
            def _():
                issue(g, lax.rem(g, RING))

        is_tail = r >= TSTART
        cn = jnp.minimum(r - t, WIN - 1)
        ct = jnp.minimum(jnp.maximum(r - TSTART, 0), TAIL - 1)
        cn16 = jnp.full((16,), cn, jnp.int32)
        ct16 = jnp.full((16,), ct, jnp.int32)
        top = plsc.load_gather(bufs.at[p], [d16a, cn16])
        bot = plsc.load_gather(bufs.at[p], [d16b, cn16])
        ttop = plsc.load_gather(tail_v, [d16a, ct16])
        tbot = plsc.load_gather(tail_v, [d16b, ct16])
        m16 = jnp.full((16,), is_tail, jnp.bool_)
        top = jnp.where(m16, ttop, top)
        bot = jnp.where(m16, tbot, bot)
        i16 = jnp.full((16,), i, jnp.int32)
        plsc.store_scatter(rows_v, [i16, d16a], top)
        plsc.store_scatter(rows_v, [i16, d16b], bot)
        return t, f

    lax.fori_loop(0, BPW, step, (jnp.int32(-1), jnp.int32(-1)))


def _gather_body(pk_hbm, orig_hbm, ta_t, tb_t, tatl, tbtl,
                 a_out, b_out,
                 pk_v, orig_v, perm_v, uniq_s, rows_v, tail_v, bufs, sems,
                 osem):
    c = lax.axis_index("c")
    s = lax.axis_index("s")
    wid = s * NC + c
    base = wid * BPW
    pltpu.sync_copy(pk_hbm.at[pl.ds(base, BPW)],
                    pk_v.at[pl.ds(0, BPW)])
    pltpu.sync_copy(orig_hbm, orig_v.at[pl.ds(0, B)])
    for k in range(BPW // 16):
        v = pk_v[pl.ds(k * 16, 16)]
        perm_v[k // 8, pl.ds((k % 8) * 16, 16)] = v & 16383
    for tab_t, tl, out in ((ta_t, tatl, a_out), (tb_t, tbtl, b_out)):
        pltpu.sync_copy(tl, tail_v)
        _gather_one(tab_t, tail_v, pk_v, orig_v, uniq_s, rows_v, bufs, sems)
        scats = []
        for j in range(NCHUNK):
            scats.append(pltpu.async_copy(
                rows_v.at[pl.ds(j * CHUNK, CHUNK)],
                out.at[perm_v.at[j]], osem))
        for sc in scats:
            sc.wait()


_f32 = jnp.float32
_padrow = jax.ShapeDtypeStruct((B, 128), _f32)


@functools.cache
def _make_gather():
  return pl.kernel(
    _gather_body,
    out_type=(_padrow, _padrow),
    mesh=plsc.VectorSubcoreMesh(core_axis_name="c", subcore_axis_name="s",
                                num_cores=NC, num_subcores=NS),
    scratch_types=[
        pltpu.VMEM((BPW + 16,), jnp.int32),
        pltpu.VMEM((B + 16,), jnp.int32),
        pltpu.VMEM((NCHUNK, CHUNK), jnp.int32),
        pltpu.SMEM((BPW + 8,), jnp.int32),
        pltpu.VMEM((BPW, 128), _f32),
        pltpu.VMEM((D, TAIL), _f32),
        pltpu.VMEM((RING, D, WIN), _f32),
        pltpu.SemaphoreType.DMA((RING,)),
        pltpu.SemaphoreType.DMA,
    ],
    compiler_params=pltpu.CompilerParams(needs_layout_passes=False),
  )


TB = 2048


def _dense_body(ug4, ig4, um4, im4,
                W0, b0, W1, b1, W2, b2, W3, b3, Wf, bf, out):
    dot = lambda a, b: lax.dot_general(
        a, b, (((1,), (0,)), ((), ())),
        precision=lax.Precision.HIGHEST, preferred_element_type=_f32)
    ug = ug4[...][:, :D]
    ig = ig4[...][:, :D]
    um = um4[...][:, :D]
    im = im4[...][:, :D]
    gmf = ug * ig
    w0 = W0[...]
    h = jnp.maximum(dot(um, w0[:D]) + dot(im, w0[D:]) + b0[...][None, :], 0.0)
    h = jnp.maximum(dot(h, W1[...]) + b1[...][None, :], 0.0)
    h = jnp.maximum(dot(h, W2[...]) + b2[...][None, :], 0.0)
    h = jnp.maximum(dot(h, W3[...]) + b3[...][None, :], 0.0)
    wf = Wf[...]
    logit = dot(gmf, wf[:D]) + dot(h, wf[D:]) + bf[...][None, :]
    out[...] = jax.nn.sigmoid(logit)


def _full(shape):
    return pl.BlockSpec(shape, lambda i: (0,) * len(shape))


_dense = pl.pallas_call(
    _dense_body,
    grid=(B // TB,),
    in_specs=[
        pl.BlockSpec((TB, 128), lambda i: (i, 0)),
        pl.BlockSpec((TB, 128), lambda i: (i, 0)),
        pl.BlockSpec((TB, 128), lambda i: (i, 0)),
        pl.BlockSpec((TB, 128), lambda i: (i, 0)),
        _full((2 * D, 64)), _full((64,)),
        _full((64, 32)), _full((32,)),
        _full((32, 16)), _full((16,)),
        _full((16, 8)), _full((8,)),
        _full((D + 8, 1)), _full((1,)),
    ],
    out_specs=pl.BlockSpec((TB, 1), lambda i: (i, 0)),
    out_shape=jax.ShapeDtypeStruct((B, 1), _f32),
)


def kernel(user_indices, item_indices, user_gmf, item_gmf, user_mlp, item_mlp,
           W0, b0, W1, b1, W2, b2, W3, b3, Wf, bf):
    ui = user_indices.astype(jnp.int32)
    ii = item_indices.astype(jnp.int32)
    pos = lax.iota(jnp.int32, B)
    g2 = _make_gather()
    upk = jnp.sort(((ui >> 7) << 14) | pos)
    ug4, um4 = g2(upk, ui, user_gmf.T, user_mlp.T,
                  user_gmf[TSTART:].T, user_mlp[TSTART:].T)
    ipk = jnp.sort(((ii >> 7) << 14) | pos)
    ig4, im4 = g2(ipk, ii, item_gmf.T, item_mlp.T,
                  item_gmf[TSTART:].T, item_mlp[TSTART:].T)
    pred = _dense(ug4, ig4, um4, im4,
                  W0, b0, W1, b1, W2, b2, W3, b3, Wf, bf)
    return pred.reshape(-1)


# speedup vs baseline: 1.0092x; 1.0092x over previous
import functools

import jax
import jax.numpy as jnp
from jax import lax
from jax.experimental import pallas as pl
from jax.experimental.pallas import tpu as pltpu
from jax.experimental.pallas import tpu_sc as plsc

B = 16384
D = 32
V = 1000000
NC = 2
NS = 16
NW = NC * NS
BPW = B // NW
CHUNK = 128
NCHUNK = BPW // CHUNK
WIN = 128
TAIL = V - (V // WIN) * WIN
TSTART = V - TAIL
RING = 8


def _win_start(r):
    return jnp.minimum((r // WIN) * WIN, TSTART - WIN)


def _gather_one(tab_t, tail_v, idx_v, uniq_s, rows_v, bufs, sems):
    d16a = lax.iota(jnp.int32, 16)
    d16b = d16a + 16

    def rd(i):
        return idx_v[pl.ds(i, 16)][0]

    def prepass(i, carry):
        prev_t, nu = carry
        t = _win_start(rd(i))
        new = t != prev_t

        @pl.when(new)
        def _():
            uniq_s[nu] = t

        return t, nu + new.astype(jnp.int32)

    _, n_u = lax.fori_loop(0, BPW, prepass, (jnp.int32(-1), jnp.int32(0)))

    def issue(f, p):
        t = uniq_s[f]
        pltpu.async_copy(
            tab_t.at[:, pl.ds(pl.multiple_of(t, WIN), WIN)],
            bufs.at[p], sems.at[p])

    for f in range(RING - 1):
        @pl.when(f < n_u)
        def _(f=f):
            issue(f, f)

    def step(i, carry):
        prev_t, f = carry
        r = rd(i)
        t = _win_start(r)
        new = t != prev_t
        f = f + new.astype(jnp.int32)
        p = lax.rem(f, RING)

        @pl.when(new)
        def _():
            pltpu.make_async_copy(
                tab_t.at[:, pl.ds(0, WIN)], bufs.at[p], sems.at[p]).wait()
            g = f + RING - 1

            @pl.when(g < n_u)
            def _():
                issue(g, lax.rem(g, RING))

        is_tail = r >= TSTART
        cn = jnp.minimum(r - t, WIN - 1)
        ct = jnp.minimum(jnp.maximum(r - TSTART, 0), TAIL - 1)
        cn16 = jnp.full((16,), cn, jnp.int32)
        ct16 = jnp.full((16,), ct, jnp.int32)
        top = plsc.load_gather(bufs.at[p], [d16a, cn16])
        bot = plsc.load_gather(bufs.at[p], [d16b, cn16])
        ttop = plsc.load_gather(tail_v, [d16a, ct16])
        tbot = plsc.load_gather(tail_v, [d16b, ct16])
        m16 = jnp.full((16,), is_tail, jnp.bool_)
        top = jnp.where(m16, ttop, top)
        bot = jnp.where(m16, tbot, bot)
        i16 = jnp.full((16,), i, jnp.int32)
        plsc.store_scatter(rows_v, [i16, d16a], top)
        plsc.store_scatter(rows_v, [i16, d16b], bot)
        return t, f

    lax.fori_loop(0, BPW, step, (jnp.int32(-1), jnp.int32(-1)))


def _gather_body(idx_hbm, perm_hbm, ta_t, tb_t, tatl, tbtl,
                 a_out, b_out,
                 idx_v, perm_v, uniq_s, rows_v, tail_v, bufs, sems, osem):
    c = lax.axis_index("c")
    s = lax.axis_index("s")
    wid = s * NC + c
    base = wid * BPW
    pltpu.sync_copy(idx_hbm.at[pl.ds(base, BPW)],
                    idx_v.at[pl.ds(0, BPW)])
    pltpu.sync_copy(perm_hbm.at[wid], perm_v)
    for tab_t, tl, out in ((ta_t, tatl, a_out), (tb_t, tbtl, b_out)):
        pltpu.sync_copy(tl, tail_v)
        _gather_one(tab_t, tail_v, idx_v, uniq_s, rows_v, bufs, sems)
        scats = []
        for j in range(NCHUNK):
            scats.append(pltpu.async_copy(
                rows_v.at[pl.ds(j * CHUNK, CHUNK)],
                out.at[perm_v.at[j]], osem))
        for sc in scats:
            sc.wait()


_f32 = jnp.float32
_padrow = jax.ShapeDtypeStruct((B, 128), _f32)


@functools.cache
def _make_gather():
  return pl.kernel(
    _gather_body,
    out_type=(_padrow, _padrow),
    mesh=plsc.VectorSubcoreMesh(core_axis_name="c", subcore_axis_name="s",
                                num_cores=NC, num_subcores=NS),
    scratch_types=[
        pltpu.VMEM((BPW + 16,), jnp.int32),
        pltpu.VMEM((NCHUNK, CHUNK), jnp.int32),
        pltpu.SMEM((BPW + 8,), jnp.int32),
        pltpu.VMEM((BPW, 128), _f32),
        pltpu.VMEM((D, TAIL), _f32),
        pltpu.VMEM((RING, D, WIN), _f32),
        pltpu.SemaphoreType.DMA((RING,)),
        pltpu.SemaphoreType.DMA,
    ],
    compiler_params=pltpu.CompilerParams(needs_layout_passes=False),
  )


TB = 2048


def _dense_body(ug4, ig4, um4, im4,
                W0, b0, W1, b1, W2, b2, W3, b3, Wf, bf, out):
    dot = lambda a, b: lax.dot_general(
        a, b, (((1,), (0,)), ((), ())),
        precision=lax.Precision.HIGHEST, preferred_element_type=_f32)
    ug = ug4[...][:, :D]
    ig = ig4[...][:, :D]
    um = um4[...][:, :D]
    im = im4[...][:, :D]
    gmf = ug * ig
    w0 = W0[...]
    h = jnp.maximum(dot(um, w0[:D]) + dot(im, w0[D:]) + b0[...][None, :], 0.0)
    h = jnp.maximum(dot(h, W1[...]) + b1[...][None, :], 0.0)
    h = jnp.maximum(dot(h, W2[...]) + b2[...][None, :], 0.0)
    h = jnp.maximum(dot(h, W3[...]) + b3[...][None, :], 0.0)
    wf = Wf[...]
    logit = dot(gmf, wf[:D]) + dot(h, wf[D:]) + bf[...][None, :]
    out[...] = jax.nn.sigmoid(logit)


def _full(shape):
    return pl.BlockSpec(shape, lambda i: (0,) * len(shape))


_dense = pl.pallas_call(
    _dense_body,
    grid=(B // TB,),
    in_specs=[
        pl.BlockSpec((TB, 128), lambda i: (i, 0)),
        pl.BlockSpec((TB, 128), lambda i: (i, 0)),
        pl.BlockSpec((TB, 128), lambda i: (i, 0)),
        pl.BlockSpec((TB, 128), lambda i: (i, 0)),
        _full((2 * D, 64)), _full((64,)),
        _full((64, 32)), _full((32,)),
        _full((32, 16)), _full((16,)),
        _full((16, 8)), _full((8,)),
        _full((D + 8, 1)), _full((1,)),
    ],
    out_specs=pl.BlockSpec((TB, 1), lambda i: (i, 0)),
    out_shape=jax.ShapeDtypeStruct((B, 1), _f32),
)


def kernel(user_indices, item_indices, user_gmf, item_gmf, user_mlp, item_mlp,
           W0, b0, W1, b1, W2, b2, W3, b3, Wf, bf):
    ui = user_indices.astype(jnp.int32)
    ii = item_indices.astype(jnp.int32)
    pos = lax.iota(jnp.int32, B)
    g2 = _make_gather()
    us, up = lax.sort_key_val(ui, pos)
    ug4, um4 = g2(us, up.reshape(NW, NCHUNK, CHUNK),
                  user_gmf.T, user_mlp.T,
                  user_gmf[TSTART:].T, user_mlp[TSTART:].T)
    its, ip = lax.sort_key_val(ii, pos)
    ig4, im4 = g2(its, ip.reshape(NW, NCHUNK, CHUNK),
                  item_gmf.T, item_mlp.T,
                  item_gmf[TSTART:].T, item_mlp[TSTART:].T)
    pred = _dense(ug4, ig4, um4, im4,
                  W0, b0, W1, b1, W2, b2, W3, b3, Wf, bf)
    return pred.reshape(-1)
